# baseline (speedup 1.0000x reference)
import jax
import jax.numpy as jnp
from jax.experimental import pallas as pl
from jax.experimental.pallas import tpu as pltpu


def _add_kernel(scale, x_ref, t_ref, o_ref):
    o_ref[...] = x_ref[...] + t_ref[...][None, :, :] * scale


def kernel(inputs, lookup_table):
    B, T, D = inputs.shape
    scale = float(D) ** 0.5
    Bb = 16
    out = pl.pallas_call(
        lambda x_ref, t_ref, o_ref: _add_kernel(scale, x_ref, t_ref, o_ref),
        grid=(B // Bb,),
        in_specs=[
            pl.BlockSpec((Bb, T, D), lambda i: (i, 0, 0)),
            pl.BlockSpec((T, D), lambda i: (0, 0)),
        ],
        out_specs=pl.BlockSpec((Bb, T, D), lambda i: (i, 0, 0)),
        out_shape=jax.ShapeDtypeStruct((B, T, D), jnp.float32),
        compiler_params=pltpu.CompilerParams(
            dimension_semantics=("parallel",),
        ),
    )(inputs, lookup_table)
    return out


# speedup vs baseline: 1.7434x; 1.7434x over previous
import jax
import jax.numpy as jnp
from jax.experimental import pallas as pl
from jax.experimental.pallas import tpu as pltpu


def _add_kernel(scale, x_ref, t_ref, o_ref):
    o_ref[...] = x_ref[...] + t_ref[...] * scale


def kernel(inputs, lookup_table):
    B, T, D = inputs.shape
    F = T * D
    scale = float(D) ** 0.5
    x = inputs.reshape(B, F)
    table = lookup_table.reshape(1, F)
    Bb = 64
    out = pl.pallas_call(
        lambda x_ref, t_ref, o_ref: _add_kernel(scale, x_ref, t_ref, o_ref),
        grid=(B // Bb,),
        in_specs=[
            pl.BlockSpec((Bb, F), lambda i: (i, 0)),
            pl.BlockSpec((1, F), lambda i: (0, 0)),
        ],
        out_specs=pl.BlockSpec((Bb, F), lambda i: (i, 0)),
        out_shape=jax.ShapeDtypeStruct((B, F), jnp.float32),
        compiler_params=pltpu.CompilerParams(
            dimension_semantics=("parallel",),
        ),
    )(x, table)
    return out.reshape(B, T, D)


# speedup vs baseline: 1.7567x; 1.0076x over previous
import math

import jax
import jax.numpy as jnp
from jax.experimental import pallas as pl
from jax.experimental.pallas import tpu as pltpu

NBUF = 4
BB = 64


def kernel(inputs, lookup_table):
    B, T, D = inputs.shape
    F = T * D
    scale = float(D) ** 0.5
    x = inputs.reshape(B, F)
    tab = lookup_table.reshape(1, F)

    mesh = pltpu.create_tensorcore_mesh("core")
    ncores = math.prod(mesh.shape.values())
    n_chunks = B // BB
    per_core = n_chunks // ncores

    def body(x_ref, t_ref, o_ref, xbuf, obuf, tbuf, insem, outsem, tsem):
        core = jax.lax.axis_index("core")
        base = core * per_core

        def in_copy(i, slot):
            return pltpu.make_async_copy(
                x_ref.at[pl.ds((base + i) * BB, BB), :], xbuf.at[slot],
                insem.at[slot],
            )

        def out_copy(i, slot):
            return pltpu.make_async_copy(
                obuf.at[slot], o_ref.at[pl.ds((base + i) * BB, BB), :],
                outsem.at[slot],
            )

        tcopy = pltpu.make_async_copy(t_ref, tbuf, tsem)
        tcopy.start()
        for k in range(NBUF):
            in_copy(k, k).start()
        tcopy.wait()
        table = tbuf[...] * scale

        def loop(i, carry):
            slot = jax.lax.rem(i, NBUF)
            in_copy(i, slot).wait()

            @pl.when(i >= NBUF)
            def _():
                out_copy(i - NBUF, slot).wait()

            obuf[slot] = xbuf[slot] + table

            out_copy(i, slot).start()

            @pl.when(i + NBUF < per_core)
            def _():
                in_copy(i + NBUF, slot).start()

            return carry

        jax.lax.fori_loop(0, per_core, loop, 0)

        for k in range(NBUF):
            i = per_core - NBUF + k
            out_copy(i, i % NBUF).wait()

    run = pl.kernel(
        body,
        out_type=jax.ShapeDtypeStruct((B, F), jnp.float32),
        mesh=mesh,
        scratch_types=[
            pltpu.VMEM((NBUF, BB, F), jnp.float32),
            pltpu.VMEM((NBUF, BB, F), jnp.float32),
            pltpu.VMEM((1, F), jnp.float32),
            pltpu.SemaphoreType.DMA((NBUF,)),
            pltpu.SemaphoreType.DMA((NBUF,)),
            pltpu.SemaphoreType.DMA,
        ],
    )
    out = run(x, tab)
    return out.reshape(B, T, D)
